# baseline (speedup 1.0000x reference)
import functools

import jax
import jax.numpy as jnp
from jax import lax
from jax.experimental import pallas as pl
from jax.experimental.pallas import tpu as pltpu
from jax.experimental.pallas import tpu_sc as plsc

_VOCAB = 32
_NEG = -1e9
_POS = 1e9


def kernel(hidden, next_ids):
    del hidden
    B, T = next_ids.shape
    N = B * T
    ids = next_ids.reshape(N).astype(jnp.int32)
    NW_CHUNKS = 4

    info = plsc.get_sparse_core_info()
    NC, NS, L = info.num_cores, info.num_subcores, info.num_lanes
    NW = NC * NS
    nper = N // NW

    mesh = plsc.VectorSubcoreMesh(core_axis_name="c", subcore_axis_name="s")

    chunk = (nper * _VOCAB) // NW_CHUNKS
    g_per_chunk = nper // (NW_CHUNKS * L)

    @functools.partial(
        pl.kernel,
        mesh=mesh,
        out_type=jax.ShapeDtypeStruct((N * _VOCAB,), jnp.float32),
        scratch_types=[
            pltpu.VMEM((nper,), jnp.int32),
            pltpu.VMEM((nper * _VOCAB,), jnp.float32),
            pltpu.SemaphoreType.DMA,
            pltpu.SemaphoreType.DMA,
        ]
        + [pltpu.SemaphoreType.DMA] * NW_CHUNKS,
        compiler_params=pltpu.CompilerParams(needs_layout_passes=False),
    )
    def sc_onehot(bg_hbm, ids_hbm, out_hbm, idx_v, buf, sem_ids, sem_out,
                  *in_sems):
        wid = lax.axis_index("s") * NC + lax.axis_index("c")
        base = wid * nper

        id_cp = pltpu.async_copy(ids_hbm.at[pl.ds(base, nper)], idx_v,
                                 sem_ids)
        in_cps = [
            pltpu.async_copy(
                bg_hbm.at[pl.ds(k * chunk, chunk)],
                buf.at[pl.ds(k * chunk, chunk)],
                in_sems[k],
            )
            for k in range(NW_CHUNKS)
        ]
        id_cp.wait()

        lane = lax.iota(jnp.int32, L)
        pos = jnp.full((L,), _POS, jnp.float32)

        def scat_body(g, c):
            tok = g * L
            v = idx_v[pl.ds(tok, L)]
            tgt = (v + 1) & (_VOCAB - 1)
            flat = (lane + tok) * _VOCAB + tgt
            plsc.store_scatter(buf, [flat], pos)
            return c

        out_cps = []
        for k in range(NW_CHUNKS):
            in_cps[k].wait()
            lax.fori_loop(k * g_per_chunk, (k + 1) * g_per_chunk,
                          scat_body, 0)
            out_cps.append(
                pltpu.async_copy(
                    buf.at[pl.ds(k * chunk, chunk)],
                    out_hbm.at[pl.ds(base * _VOCAB + k * chunk, chunk)],
                    sem_out,
                )
            )
        for cp in out_cps:
            cp.wait()

    bg = jnp.full((nper * _VOCAB,), _NEG, jnp.float32)
    out = sc_onehot(bg, ids)
    return out.reshape(B, T, _VOCAB)


# speedup vs baseline: 1.1102x; 1.1102x over previous
import functools

import jax
import jax.numpy as jnp
from jax import lax
from jax.experimental import pallas as pl
from jax.experimental.pallas import tpu as pltpu
from jax.experimental.pallas import tpu_sc as plsc

_VOCAB = 32
_NEG = -1e9
_POS = 1e9


def kernel(hidden, next_ids):
    del hidden
    B, T = next_ids.shape
    N = B * T
    ids = next_ids.reshape(N).astype(jnp.int32)
    NW_CHUNKS = 4

    info = plsc.get_sparse_core_info()
    NC, NS, L = info.num_cores, info.num_subcores, info.num_lanes
    NW = NC * NS
    nper = N // NW

    mesh = plsc.VectorSubcoreMesh(core_axis_name="c", subcore_axis_name="s")

    chunk = (nper * _VOCAB) // NW_CHUNKS
    g_per_chunk = nper // (NW_CHUNKS * L)

    @functools.partial(
        pl.kernel,
        mesh=mesh,
        out_type=jax.ShapeDtypeStruct((N * _VOCAB,), jnp.float32),
        scratch_types=[
            pltpu.VMEM((nper,), jnp.int32),
            pltpu.VMEM((nper * _VOCAB,), jnp.float32),
            pltpu.SemaphoreType.DMA,
            pltpu.SemaphoreType.DMA,
        ],
        compiler_params=pltpu.CompilerParams(needs_layout_passes=False),
    )
    def sc_onehot(ids_hbm, out_hbm, idx_v, buf, sem_ids, sem_out):
        wid = lax.axis_index("s") * NC + lax.axis_index("c")
        base = wid * nper

        id_cp = pltpu.async_copy(ids_hbm.at[pl.ds(base, nper)], idx_v,
                                 sem_ids)

        neg = jnp.full((L,), _NEG, jnp.float32)
        lane = lax.iota(jnp.int32, L)
        pos = jnp.full((L,), _POS, jnp.float32)

        def init_body(i, c):
            for u in range(16):
                buf[pl.ds((i * 16 + u) * L, L)] = neg
            return c

        def scat_body(g, c):
            tok = g * L
            v = idx_v[pl.ds(tok, L)]
            tgt = (v + 1) & (_VOCAB - 1)
            flat = (lane + tok) * _VOCAB + tgt
            plsc.store_scatter(buf, [flat], pos)
            return c

        i_per_chunk = chunk // (16 * L)
        out_cps = []
        for k in range(NW_CHUNKS):
            lax.fori_loop(k * i_per_chunk, (k + 1) * i_per_chunk,
                          init_body, 0)
            if k == 0:
                id_cp.wait()
            lax.fori_loop(k * g_per_chunk, (k + 1) * g_per_chunk,
                          scat_body, 0)
            out_cps.append(
                pltpu.async_copy(
                    buf.at[pl.ds(k * chunk, chunk)],
                    out_hbm.at[pl.ds(base * _VOCAB + k * chunk, chunk)],
                    sem_out,
                )
            )
        for cp in out_cps:
            cp.wait()

    out = sc_onehot(ids)
    return out.reshape(B, T, _VOCAB)
